# Initial kernel scaffold; baseline (speedup 1.0000x reference)
import jax
import jax.numpy as jnp
from jax.experimental import pallas as pl


def kernel(x, weight, bias):
    raise NotImplementedError("write your pallas kernel here")


# speedup vs baseline: 1.8567x; 1.8567x over previous
import functools

import jax
import jax.numpy as jnp
from jax.experimental import pallas as pl
from jax.experimental.pallas import tpu as pltpu


def _conv3x3_kernel(x_ref, w_ref, b_ref, o_ref, *, C, H, W):
    HW = H * W
    xs = x_ref[0].astype(jnp.bfloat16)
    P = W + 1
    padded = jnp.pad(xs, ((0, 0), (P, P)))
    col = jax.lax.broadcasted_iota(jnp.int32, (C, HW), 1) % W

    taps = []
    for kh in (0, 1, 2):
        for kw in (0, 1, 2):
            s = (kh - 1) * W + (kw - 1)
            t = padded[:, P + s: P + s + HW]
            if kw == 0:
                t = jnp.where(col != 0, t, 0)
            elif kw == 2:
                t = jnp.where(col != W - 1, t, 0)
            taps.append(t)
    patches = jnp.concatenate(taps, axis=0)

    acc = jnp.dot(w_ref[...], patches,
                  preferred_element_type=jnp.float32)
    acc = acc + b_ref[...]
    o_ref[0] = jnp.maximum(acc, 0.0).astype(o_ref.dtype)


def kernel(x, weight, bias):
    N, C, H, W = x.shape
    Cout = weight.shape[0]
    HW = H * W
    K = 9 * C

    x3 = x.reshape(N, C, HW)
    wf = jnp.transpose(weight, (0, 2, 3, 1)).reshape(Cout, K).astype(jnp.bfloat16)
    b2 = bias.astype(jnp.float32).reshape(Cout, 1)

    out = pl.pallas_call(
        functools.partial(_conv3x3_kernel, C=C, H=H, W=W),
        out_shape=jax.ShapeDtypeStruct((N, Cout, HW), x.dtype),
        grid=(N,),
        in_specs=[
            pl.BlockSpec((1, C, HW), lambda n: (n, 0, 0)),
            pl.BlockSpec((Cout, K), lambda n: (0, 0)),
            pl.BlockSpec((Cout, 1), lambda n: (0, 0)),
        ],
        out_specs=pl.BlockSpec((1, Cout, HW), lambda n: (n, 0, 0)),
        compiler_params=pltpu.CompilerParams(
            dimension_semantics=("parallel",),
            vmem_limit_bytes=64 * 1024 * 1024,
        ),
    )(x3, wf, b2)
    return out.reshape(N, Cout, H, W)
